# baseline (speedup 1.0000x reference)
import jax
import jax.numpy as jnp
from jax.experimental import pallas as pl
from jax.experimental.pallas import tpu as pltpu

_TILE = 1024
_LANES = 128


def _tok_kernel(m_ref, s_ref, h_ref, btn_ref, keys_ref, yp_ref, gui_ref,
                mouse_ref, scroll_ref, hotbar_ref, slot_ref,
                bW_ref, bb_ref, kW_ref, kb_ref, ygW_ref, ygb_ref, out_ref):
    f32 = jnp.float32
    rows = _TILE // _LANES
    i = pl.program_id(0)

    def dot(a, b):
        return jnp.dot(a, b, preferred_element_type=f32)

    def iota(n):
        return jax.lax.broadcasted_iota(jnp.int32, (1, n), 1)

    def col(ref):
        blk = ref[pl.ds(i * rows, rows), :]
        t = jnp.transpose(blk)
        parts = [t[:, r:r + 1] for r in range(rows)]
        return jnp.concatenate(parts, axis=0)

    m = col(m_ref)
    oh0 = (m == iota(121)).astype(f32)
    tok0 = dot(oh0, mouse_ref[...])
    tok0 = tok0 + slot_ref[0:1, :]

    s = col(s_ref)
    oh1 = (s == iota(3)).astype(f32)
    f1 = jnp.concatenate([oh1, btn_ref[...]], axis=1)
    w1 = jnp.concatenate([scroll_ref[...], bW_ref[...]], axis=0)
    tok1 = dot(f1, w1)
    tok1 = tok1 + (slot_ref[1:2, :] + bb_ref[...])

    tok2 = dot(keys_ref[...], kW_ref[...])
    tok2 = tok2 + (slot_ref[2:3, :] + kb_ref[...])

    h = col(h_ref)
    oh3 = (h == iota(9)).astype(f32)
    f3 = jnp.concatenate([oh3, yp_ref[...], gui_ref[...]], axis=1)
    w3 = jnp.concatenate([hotbar_ref[...], ygW_ref[...]], axis=0)
    tok3 = dot(f3, w3)
    tok3 = tok3 + (slot_ref[3:4, :] + ygb_ref[...])

    out_ref[:, 0, :] = tok0
    out_ref[:, 1, :] = tok1
    out_ref[:, 2, :] = tok2
    out_ref[:, 3, :] = tok3


def kernel(mouse_cat, scroll, buttons, keys, yaw_pitch, gui, hotbar,
           mouse_table, scroll_table, hotbar_table, slot_table,
           buttons_W, buttons_b, keys_W, keys_b, yawgui_W, yawgui_b):
    B, T = mouse_cat.shape
    D = mouse_table.shape[1]
    N = B * T

    m = mouse_cat.reshape(N // _LANES, _LANES).astype(jnp.int32)
    s = scroll.reshape(N // _LANES, _LANES).astype(jnp.int32)
    h = hotbar.reshape(N // _LANES, _LANES).astype(jnp.int32)
    btn = buttons.reshape(N, 3)
    ky = keys.reshape(N, keys.shape[-1])
    yp = yaw_pitch.reshape(N, 2)
    gu = gui.reshape(N, 2)
    bb = buttons_b.reshape(1, D)
    kb = keys_b.reshape(1, D)
    ygb = yawgui_b.reshape(1, D)

    grid = (N // _TILE,)

    def tok_map(i):
        return (i, 0)

    def full_map(i):
        return (0, 0)

    out = pl.pallas_call(
        _tok_kernel,
        grid=grid,
        in_specs=[
            pl.BlockSpec(m.shape, full_map),
            pl.BlockSpec(s.shape, full_map),
            pl.BlockSpec(h.shape, full_map),
            pl.BlockSpec((_TILE, 3), tok_map),
            pl.BlockSpec((_TILE, ky.shape[1]), tok_map),
            pl.BlockSpec((_TILE, 2), tok_map),
            pl.BlockSpec((_TILE, 2), tok_map),
            pl.BlockSpec(mouse_table.shape, full_map),
            pl.BlockSpec(scroll_table.shape, full_map),
            pl.BlockSpec(hotbar_table.shape, full_map),
            pl.BlockSpec(slot_table.shape, full_map),
            pl.BlockSpec(buttons_W.shape, full_map),
            pl.BlockSpec((1, D), full_map),
            pl.BlockSpec(keys_W.shape, full_map),
            pl.BlockSpec((1, D), full_map),
            pl.BlockSpec(yawgui_W.shape, full_map),
            pl.BlockSpec((1, D), full_map),
        ],
        out_specs=pl.BlockSpec((_TILE, 4, D), lambda i: (i, 0, 0)),
        out_shape=jax.ShapeDtypeStruct((N, 4, D), jnp.float32),
        compiler_params=pltpu.CompilerParams(
            dimension_semantics=("parallel",),
        ),
    )(m, s, h, btn, ky, yp, gu, mouse_table, scroll_table, hotbar_table,
      slot_table, buttons_W, bb, keys_W, kb, yawgui_W, ygb)

    return out.reshape(B, T, 4, D)


# speedup vs baseline: 1.1472x; 1.1472x over previous
import jax
import jax.numpy as jnp
from jax.experimental import pallas as pl
from jax.experimental.pallas import tpu as pltpu

_TILE = 1024


def _tok_kernel(feat_ref,
                mouse_ref, scroll_ref, hotbar_ref, slot_ref,
                bW_ref, bb_ref, kW_ref, kb_ref, ygW_ref, ygb_ref, out_ref):
    feat = feat_ref[...]
    f32 = jnp.float32

    def dot(a, b):
        return jnp.dot(a, b, preferred_element_type=f32)

    def iota(n):
        return jax.lax.broadcasted_iota(jnp.int32, (1, n), 1)

    m = feat[:, 0:1].astype(jnp.int32)
    oh0 = (m == iota(121)).astype(f32)
    w0 = mouse_ref[...] + slot_ref[0:1, :]
    tok0 = dot(oh0, w0)

    s = feat[:, 1:2].astype(jnp.int32)
    oh1 = (s == iota(3)).astype(f32)
    f1 = jnp.concatenate([oh1, feat[:, 3:6]], axis=1)
    w1 = jnp.concatenate(
        [scroll_ref[...] + (slot_ref[1:2, :] + bb_ref[...]), bW_ref[...]],
        axis=0)
    tok1 = dot(f1, w1)

    ones = jnp.ones((feat.shape[0], 1), f32)
    f2 = jnp.concatenate([feat[:, 6:29], ones], axis=1)
    w2 = jnp.concatenate(
        [kW_ref[...], slot_ref[2:3, :] + kb_ref[...]], axis=0)
    tok2 = dot(f2, w2)

    h = feat[:, 2:3].astype(jnp.int32)
    oh3 = (h == iota(9)).astype(f32)
    f3 = jnp.concatenate([oh3, feat[:, 29:33]], axis=1)
    w3 = jnp.concatenate(
        [hotbar_ref[...] + (slot_ref[3:4, :] + ygb_ref[...]), ygW_ref[...]],
        axis=0)
    tok3 = dot(f3, w3)

    out_ref[:, 0, :] = tok0
    out_ref[:, 1, :] = tok1
    out_ref[:, 2, :] = tok2
    out_ref[:, 3, :] = tok3


def kernel(mouse_cat, scroll, buttons, keys, yaw_pitch, gui, hotbar,
           mouse_table, scroll_table, hotbar_table, slot_table,
           buttons_W, buttons_b, keys_W, keys_b, yawgui_W, yawgui_b):
    B, T = mouse_cat.shape
    D = mouse_table.shape[1]
    N = B * T
    f32 = jnp.float32

    feat = jnp.concatenate(
        [mouse_cat[..., None].astype(f32),
         scroll[..., None].astype(f32),
         hotbar[..., None].astype(f32),
         buttons, keys, yaw_pitch, gui], axis=-1).reshape(N, 33)
    bb = buttons_b.reshape(1, D)
    kb = keys_b.reshape(1, D)
    ygb = yawgui_b.reshape(1, D)

    grid = (N // _TILE,)

    def full_map(i):
        return (0, 0)

    out = pl.pallas_call(
        _tok_kernel,
        grid=grid,
        in_specs=[
            pl.BlockSpec((_TILE, 33), lambda i: (i, 0)),
            pl.BlockSpec(mouse_table.shape, full_map),
            pl.BlockSpec(scroll_table.shape, full_map),
            pl.BlockSpec(hotbar_table.shape, full_map),
            pl.BlockSpec(slot_table.shape, full_map),
            pl.BlockSpec(buttons_W.shape, full_map),
            pl.BlockSpec((1, D), full_map),
            pl.BlockSpec(keys_W.shape, full_map),
            pl.BlockSpec((1, D), full_map),
            pl.BlockSpec(yawgui_W.shape, full_map),
            pl.BlockSpec((1, D), full_map),
        ],
        out_specs=pl.BlockSpec((_TILE, 4, D), lambda i: (i, 0, 0)),
        out_shape=jax.ShapeDtypeStruct((N, 4, D), jnp.float32),
        compiler_params=pltpu.CompilerParams(
            dimension_semantics=("parallel",),
        ),
    )(feat, mouse_table, scroll_table, hotbar_table, slot_table,
      buttons_W, bb, keys_W, kb, yawgui_W, ygb)

    return out.reshape(B, T, 4, D)
